# Initial kernel scaffold; baseline (speedup 1.0000x reference)
import functools

import jax
import jax.numpy as jnp
from jax import lax
from jax.experimental import pallas as pl
from jax.experimental.pallas import tpu as pltpu
from jax.experimental.pallas import tpu_sc as plsc

N = 10000
E = 320000
D_IN = 128
H = 16

NC = 2
NS = 16
NW = NC * NS
CHUNK = 128
NB = -(-E // (NW * CHUNK))
E_PAD = NW * NB * CHUNK
N_ACC = 10240

_f32 = jnp.float32


def _make_edge_pass(D):
    mesh = plsc.VectorSubcoreMesh(core_axis_name="c", subcore_axis_name="s")
    rows_per_sub = N_ACC // NS

    @functools.partial(
        pl.kernel,
        mesh=mesh,
        out_type=jax.ShapeDtypeStruct((NC, N_ACC, D), _f32),
        scratch_types=[
            pltpu.VMEM((NB, CHUNK), jnp.int32),
            pltpu.VMEM((NB, CHUNK), jnp.int32),
            pltpu.VMEM((CHUNK, D), _f32),
            pltpu.VMEM_SHARED((N_ACC, D), _f32),
        ],
    )
    def edge_pass(table_hbm, src_hbm, dst_hbm, zero_hbm, out_hbm,
                  src_v, dst_v, rows_v, acc_sh):
        c = lax.axis_index("c")
        s = lax.axis_index("s")
        w = c * NS + s
        r0 = s * rows_per_sub
        pltpu.sync_copy(zero_hbm.at[pl.ds(r0, rows_per_sub)],
                        acc_sh.at[pl.ds(r0, rows_per_sub)])
        pltpu.sync_copy(src_hbm.at[w], src_v)
        pltpu.sync_copy(dst_hbm.at[w], dst_v)
        plsc.subcore_barrier()

        @pl.loop(0, NB)
        def _(j):
            pltpu.sync_copy(table_hbm.at[src_v.at[j]], rows_v)
            pltpu.sync_copy(rows_v, acc_sh.at[dst_v.at[j]], add=True)

        plsc.subcore_barrier()
        pltpu.sync_copy(acc_sh.at[pl.ds(r0, rows_per_sub)],
                        out_hbm.at[c, pl.ds(r0, rows_per_sub)])

    return edge_pass


_edge_pass_32 = _make_edge_pass(2 * H)
_edge_pass_16 = _make_edge_pass(H)


def _pre_body(x_ref, wl_ref, wr_ref, b_ref, t_ref, r_ref):
    xb = x_ref[...]
    y = jnp.dot(xb, wl_ref[...], preferred_element_type=_f32)
    t_ref[...] = jnp.concatenate([y, jnp.ones_like(y)], axis=1)
    r_ref[...] = jnp.dot(xb, wr_ref[...], preferred_element_type=_f32) + b_ref[...]


_pre = pl.pallas_call(
    _pre_body,
    out_shape=(jax.ShapeDtypeStruct((N, 2 * H), _f32),
               jax.ShapeDtypeStruct((N, H), _f32)),
)


def _mid1_body(p0_ref, p1_ref, r_ref, wl_ref, wr_ref, b_ref,
               y_ref, rn_ref, inv_ref):
    p = p0_ref[...] + p1_ref[...]
    inv = 1.0 / jnp.maximum(p[:, H:], 1.0)
    h = p[:, :H] * inv + r_ref[...]
    y_ref[...] = jnp.dot(h, wl_ref[...], preferred_element_type=_f32)
    rn_ref[...] = jnp.dot(h, wr_ref[...], preferred_element_type=_f32) + b_ref[...]
    inv_ref[...] = inv


_mid1 = pl.pallas_call(
    _mid1_body,
    out_shape=(jax.ShapeDtypeStruct((N, H), _f32),
               jax.ShapeDtypeStruct((N, H), _f32),
               jax.ShapeDtypeStruct((N, H), _f32)),
)


def _mid2_body(p0_ref, p1_ref, inv_ref, r_ref, wl_ref, wr_ref, b_ref,
               y_ref, rn_ref):
    h = (p0_ref[...] + p1_ref[...]) * inv_ref[...] + r_ref[...]
    y_ref[...] = jnp.dot(h, wl_ref[...], preferred_element_type=_f32)
    rn_ref[...] = jnp.dot(h, wr_ref[...], preferred_element_type=_f32) + b_ref[...]


_mid2 = pl.pallas_call(
    _mid2_body,
    out_shape=(jax.ShapeDtypeStruct((N, H), _f32),
               jax.ShapeDtypeStruct((N, H), _f32)),
)


def _final_body(p0_ref, p1_ref, inv_ref, r_ref, wlin_ref, blin_ref, o_ref):
    h = (p0_ref[...] + p1_ref[...]) * inv_ref[...] + r_ref[...]
    pooled = jnp.sum(h, axis=0, keepdims=True) * (1.0 / N)
    o_ref[...] = jnp.dot(pooled, wlin_ref[...], preferred_element_type=_f32) \
        + blin_ref[...]


_final = pl.pallas_call(
    _final_body,
    out_shape=jax.ShapeDtypeStruct((1, 1), _f32),
)


def kernel(x, edge_index, Wl0, Wr0, b0, Wl1, Wr1, b1, Wl2, Wr2, b2, Wlin, blin):
    src = edge_index[0]
    dst = edge_index[1]
    pad = E_PAD - E
    src3 = jnp.concatenate([src, jnp.zeros((pad,), jnp.int32)]).reshape(NW, NB, CHUNK)
    dst3 = jnp.concatenate([dst, jnp.full((pad,), N, jnp.int32)]).reshape(NW, NB, CHUNK)
    zero32 = jnp.zeros((N_ACC, 2 * H), _f32)
    zero16 = jnp.zeros((N_ACC, H), _f32)

    table0, r0 = _pre(x, Wl0, Wr0, b0.reshape(1, H))
    pa = _edge_pass_32(table0, src3, dst3, zero32)
    y1, r1, inv = _mid1(pa[0, :N], pa[1, :N], r0, Wl1, Wr1, b1.reshape(1, H))
    pb = _edge_pass_16(y1, src3, dst3, zero16)
    y2, r2 = _mid2(pb[0, :N], pb[1, :N], inv, r1, Wl2, Wr2, b2.reshape(1, H))
    pc = _edge_pass_16(y2, src3, dst3, zero16)
    return _final(pc[0, :N], pc[1, :N], inv, r2, Wlin, blin.reshape(1, 1))


# speedup vs baseline: 11.3817x; 11.3817x over previous
import functools

import jax
import jax.numpy as jnp
from jax import lax
from jax.experimental import pallas as pl
from jax.experimental.pallas import tpu as pltpu
from jax.experimental.pallas import tpu_sc as plsc

N = 10000
E = 320000
D_IN = 128
H = 16

NC = 2
NS = 16
NW = NC * NS
CHUNK = 128
NB = -(-E // (NW * CHUNK))
E_PAD = NW * NB * CHUNK
N_ACC = 10240

_f32 = jnp.float32


def _make_edge_pass(D):
    mesh = plsc.VectorSubcoreMesh(core_axis_name="c", subcore_axis_name="s")
    rows_per_sub = N_ACC // NS

    @functools.partial(
        pl.kernel,
        mesh=mesh,
        out_type=jax.ShapeDtypeStruct((NC, N_ACC, D), _f32),
        compiler_params=pltpu.CompilerParams(use_tc_tiling_on_sc=False),
        scratch_types=[
            pltpu.VMEM((NB, CHUNK), jnp.int32),
            pltpu.VMEM((NB, CHUNK), jnp.int32),
            pltpu.VMEM((CHUNK, D), _f32),
            pltpu.VMEM_SHARED((N_ACC, D), _f32),
        ],
    )
    def edge_pass(table_hbm, src_hbm, dst_hbm, zero_hbm, out_hbm,
                  src_v, dst_v, rows_v, acc_sh):
        c = lax.axis_index("c")
        s = lax.axis_index("s")
        w = c * NS + s
        r0 = s * rows_per_sub
        pltpu.sync_copy(zero_hbm.at[pl.ds(r0, rows_per_sub)],
                        acc_sh.at[pl.ds(r0, rows_per_sub)])
        pltpu.sync_copy(src_hbm.at[w], src_v)
        pltpu.sync_copy(dst_hbm.at[w], dst_v)
        plsc.subcore_barrier()

        @pl.loop(0, NB)
        def _(j):
            pltpu.sync_copy(table_hbm.at[src_v.at[j]], rows_v)
            pltpu.sync_copy(rows_v, acc_sh.at[dst_v.at[j]], add=True)

        plsc.subcore_barrier()
        pltpu.sync_copy(acc_sh.at[pl.ds(r0, rows_per_sub)],
                        out_hbm.at[c, pl.ds(r0, rows_per_sub)])

    return edge_pass


_edge_pass_32 = _make_edge_pass(2 * H)
_edge_pass_16 = _make_edge_pass(H)


def _pre_body(x_ref, wl_ref, wr_ref, b_ref, t_ref, r_ref):
    xb = x_ref[...]
    y = jnp.dot(xb, wl_ref[...], preferred_element_type=_f32)
    t_ref[...] = jnp.concatenate([y, jnp.ones_like(y)], axis=1)
    r_ref[...] = jnp.dot(xb, wr_ref[...], preferred_element_type=_f32) + b_ref[...]


_pre = pl.pallas_call(
    _pre_body,
    out_shape=(jax.ShapeDtypeStruct((N, 2 * H), _f32),
               jax.ShapeDtypeStruct((N, H), _f32)),
)


def _mid1_body(p0_ref, p1_ref, r_ref, wl_ref, wr_ref, b_ref,
               y_ref, rn_ref, inv_ref):
    p = p0_ref[...] + p1_ref[...]
    inv = 1.0 / jnp.maximum(p[:, H:], 1.0)
    h = p[:, :H] * inv + r_ref[...]
    y_ref[...] = jnp.dot(h, wl_ref[...], preferred_element_type=_f32)
    rn_ref[...] = jnp.dot(h, wr_ref[...], preferred_element_type=_f32) + b_ref[...]
    inv_ref[...] = inv


_mid1 = pl.pallas_call(
    _mid1_body,
    out_shape=(jax.ShapeDtypeStruct((N, H), _f32),
               jax.ShapeDtypeStruct((N, H), _f32),
               jax.ShapeDtypeStruct((N, H), _f32)),
)


def _mid2_body(p0_ref, p1_ref, inv_ref, r_ref, wl_ref, wr_ref, b_ref,
               y_ref, rn_ref):
    h = (p0_ref[...] + p1_ref[...]) * inv_ref[...] + r_ref[...]
    y_ref[...] = jnp.dot(h, wl_ref[...], preferred_element_type=_f32)
    rn_ref[...] = jnp.dot(h, wr_ref[...], preferred_element_type=_f32) + b_ref[...]


_mid2 = pl.pallas_call(
    _mid2_body,
    out_shape=(jax.ShapeDtypeStruct((N, H), _f32),
               jax.ShapeDtypeStruct((N, H), _f32)),
)


def _final_body(p0_ref, p1_ref, inv_ref, r_ref, wlin_ref, blin_ref, o_ref):
    h = (p0_ref[...] + p1_ref[...]) * inv_ref[...] + r_ref[...]
    pooled = jnp.sum(h, axis=0, keepdims=True) * (1.0 / N)
    o_ref[...] = jnp.dot(pooled, wlin_ref[...], preferred_element_type=_f32) \
        + blin_ref[...]


_final = pl.pallas_call(
    _final_body,
    out_shape=jax.ShapeDtypeStruct((1, 1), _f32),
)


def kernel(x, edge_index, Wl0, Wr0, b0, Wl1, Wr1, b1, Wl2, Wr2, b2, Wlin, blin):
    src = edge_index[0]
    dst = edge_index[1]
    pad = E_PAD - E
    src3 = jnp.concatenate([src, jnp.zeros((pad,), jnp.int32)]).reshape(NW, NB, CHUNK)
    dst3 = jnp.concatenate([dst, jnp.full((pad,), N, jnp.int32)]).reshape(NW, NB, CHUNK)
    zero32 = jnp.zeros((N_ACC, 2 * H), _f32)
    zero16 = jnp.zeros((N_ACC, H), _f32)

    table0, r0 = _pre(x, Wl0, Wr0, b0.reshape(1, H))
    pa = _edge_pass_32(table0, src3, dst3, zero32)
    y1, r1, inv = _mid1(pa[0, :N], pa[1, :N], r0, Wl1, Wr1, b1.reshape(1, H))
    pb = _edge_pass_16(y1, src3, dst3, zero16)
    y2, r2 = _mid2(pb[0, :N], pb[1, :N], inv, r1, Wl2, Wr2, b2.reshape(1, H))
    pc = _edge_pass_16(y2, src3, dst3, zero16)
    return _final(pc[0, :N], pc[1, :N], inv, r2, Wlin, blin.reshape(1, 1))
